# baseline (speedup 1.0000x reference)
import jax
import jax.numpy as jnp
from jax import lax
from jax.experimental import pallas as pl
from jax.experimental.pallas import tpu as pltpu
from jax.experimental.pallas import tpu_sc as plsc

B, S = 1024, 200
BS = B * S
HALF = BS // 2
INTD = 64
GW = 128
HD = 192
REP = 64


_NC, _NS = 2, 16
_NW = _NC * _NS
_C = 64


def _make_sc_body(n_tab, src_base, n_pos):
    per_w = n_pos // _NW
    nchunk = per_w // _C

    def body(*refs):
        idxs = refs[:n_tab]
        tabs = refs[n_tab:2 * n_tab]
        outs = refs[2 * n_tab:3 * n_tab]
        ivs = refs[3 * n_tab:4 * n_tab]
        bufs = (refs[4 * n_tab:5 * n_tab], refs[5 * n_tab:6 * n_tab])
        sems = refs[6 * n_tab:6 * n_tab + 2]

        wid = lax.axis_index("s") * _NC + lax.axis_index("c")
        src0 = src_base + wid * per_w
        dst0 = wid * per_w

        for j in range(n_tab):
            pltpu.sync_copy(idxs[j].at[pl.ds(src0, per_w)], ivs[j])

        def fire(g, s):
            for j in range(n_tab):
                pltpu.async_copy(tabs[j].at[ivs[j].at[pl.ds(g * _C, _C)]],
                                 bufs[s][j], sems[s])

        def drain(s):
            for j in range(n_tab):
                pltpu.make_async_copy(tabs[j].at[pl.ds(0, _C)],
                                      bufs[s][j], sems[s]).wait()

        def scatter(g, s):
            base = dst0 + g * _C
            for j in range(n_tab):
                pltpu.sync_copy(bufs[s][j], outs[j].at[pl.ds(base, _C)])

        fire(0, 0)

        def pair(k, _):
            g = 2 * k
            fire(g + 1, 1)
            drain(0)
            scatter(g, 0)
            fire(g + 2, 0)
            drain(1)
            scatter(g + 1, 1)
            return ()

        lax.fori_loop(0, nchunk // 2 - 1, pair, (), unroll=False)
        g = nchunk - 2
        fire(g + 1, 1)
        drain(0)
        scatter(g, 0)
        drain(1)
        scatter(g + 1, 1)

    return body


def _sc_gather(idxs, tabs, src_base, n_pos):
    n_tab = len(tabs)
    per_w = n_pos // _NW
    mesh = plsc.VectorSubcoreMesh(core_axis_name="c", subcore_axis_name="s")
    row = jax.ShapeDtypeStruct((n_pos, GW), jnp.float32)
    ebuf = pltpu.VMEM((_C, GW), jnp.float32)
    f = pl.kernel(
        _make_sc_body(n_tab, src_base, n_pos),
        mesh=mesh,
        out_type=tuple([row] * n_tab),
        scratch_types=(
            [pltpu.VMEM((per_w,), jnp.int32)] * n_tab
            + [ebuf] * (2 * n_tab)
            + [pltpu.SemaphoreType.DMA, pltpu.SemaphoreType.DMA]
        ),
    )
    out = f(*idxs, *tabs)
    return out if isinstance(out, (tuple, list)) else (out,)


_R = 2048
_NBLK = HALF // _R


def _tc_body(c1_ref, c2_ref, c3_ref, i0_ref, el_ref, du_ref, emb0_ref,
             w0_ref, w_ref, wel_ref, wdu_ref, b_ref, *rest):
    out_ref = rest[-1]
    w = w_ref[...]
    acc = jnp.dot(c1_ref[...], w[0 * GW:1 * GW],
                  preferred_element_type=jnp.float32)
    acc += jnp.dot(c2_ref[...], w[1 * GW:2 * GW],
                   preferred_element_type=jnp.float32)
    acc += jnp.dot(c3_ref[...], w[2 * GW:3 * GW],
                   preferred_element_type=jnp.float32)
    m0 = jnp.dot(emb0_ref[...], w0_ref[...],
                 preferred_element_type=jnp.float32)
    iota8 = lax.broadcasted_iota(jnp.int32, (1, 8), 1)
    oh = jnp.where(i0_ref[...][:, None] == iota8, 1.0, 0.0)
    acc += jnp.dot(oh, m0, preferred_element_type=jnp.float32)
    el = el_ref[...][:, None]
    du = du_ref[...][:, None]
    out_ref[...] = (acc + el * wel_ref[...][None, :] + du * wdu_ref[...][None, :]
                    + b_ref[...][None, :])


def _tc_matmul(half, c1, c2, c3, i0, el, du, emb0, w0, w_mid, w_el, w_du, b,
               xprev=None):
    off = half * _NBLK
    half_spec = pl.BlockSpec((_R, GW), lambda i: (i, 0))
    full_spec = pl.BlockSpec((_R, GW), lambda i, off=off: (i + off, 0))
    flat_spec = pl.BlockSpec((_R,), lambda i, off=off: (i + off,))
    in_specs = [
        half_spec, full_spec, full_spec,
        flat_spec, flat_spec, flat_spec,
        pl.BlockSpec((8, INTD), lambda i: (0, 0)),
        pl.BlockSpec((INTD, HD), lambda i: (0, 0)),
        pl.BlockSpec((3 * GW, HD), lambda i: (0, 0)),
        pl.BlockSpec((HD,), lambda i: (0,)),
        pl.BlockSpec((HD,), lambda i: (0,)),
        pl.BlockSpec((HD,), lambda i: (0,)),
    ]
    args = [c1, c2, c3, i0, el, du, emb0, w0, w_mid, w_el, w_du, b]
    aliases = {}
    if xprev is not None:
        in_specs.append(pl.BlockSpec(memory_space=pl.ANY))
        args.append(xprev)
        aliases = {12: 0}
    return pl.pallas_call(
        _tc_body,
        grid=(_NBLK,),
        in_specs=in_specs,
        out_specs=pl.BlockSpec((_R, HD), lambda i, off=off: (i + off, 0)),
        out_shape=jax.ShapeDtypeStruct((BS, HD), jnp.float32),
        input_output_aliases=aliases,
    )(*args)


def kernel(interaction, assessmentItemID, testId, KnowledgeTag, elapsed,
           duration, emb_interaction, emb_assessmentItemID, emb_testId,
           emb_KnowledgeTag, W, b):
    batch_size, seq_len = interaction.shape[0], interaction.shape[1]
    zcol = jnp.zeros((100001, GW - INTD), jnp.float32)
    t1 = jnp.concatenate([emb_assessmentItemID, zcol], axis=1)
    rep = lambda t: jnp.tile(jnp.pad(t, ((0, 1024 - 1001), (0, GW - INTD))),
                             (REP, 1))
    t2 = rep(emb_testId)
    t3 = rep(emb_KnowledgeTag)
    iota = jnp.arange(BS, dtype=jnp.int32)
    spread = (iota & (REP - 1)) << 10
    i1 = assessmentItemID.reshape(-1)
    i2 = testId.reshape(-1) + spread
    i3 = KnowledgeTag.reshape(-1) + spread
    c2, c3 = _sc_gather((i2, i3), (t2, t3), 0, BS)
    (c1a,) = _sc_gather((i1,), (t1,), 0, HALF)
    (c1b,) = _sc_gather((i1,), (t1,), HALF, HALF)
    emb0 = jnp.pad(emb_interaction, ((0, 8 - 3), (0, 0)))
    w_pad = jnp.concatenate(
        [W[INTD:4 * INTD].reshape(3, INTD, HD),
         jnp.zeros((3, GW - INTD, HD), jnp.float32)], axis=1).reshape(3 * GW, HD)
    i0 = interaction.reshape(-1)
    el = elapsed.reshape(-1)
    du = duration.reshape(-1)
    common = (i0, el, du, emb0, W[:INTD], w_pad,
              W[4 * INTD], W[4 * INTD + 1], b)
    xa = _tc_matmul(0, c1a, c2, c3, *common)
    x = _tc_matmul(1, c1b, c2, c3, *common, xprev=xa)
    return (x.reshape(batch_size, seq_len, HD), batch_size, seq_len)


# speedup vs baseline: 1.0264x; 1.0264x over previous
import jax
import jax.numpy as jnp
from jax import lax
from jax.experimental import pallas as pl
from jax.experimental.pallas import tpu as pltpu
from jax.experimental.pallas import tpu_sc as plsc

B, S = 1024, 200
BS = B * S
HALF = BS // 2
INTD = 64
GW = 128
HD = 192
REP = 64


_NC, _NS = 2, 16
_NW = _NC * _NS
_C = 64


def _make_sc_body(n_tab, src_base, n_pos):
    per_w = n_pos // _NW
    nchunk = per_w // _C

    def body(*refs):
        idxs = refs[:n_tab]
        tabs = refs[n_tab:2 * n_tab]
        outs = refs[2 * n_tab:3 * n_tab]
        ivs = refs[3 * n_tab:4 * n_tab]
        bufs = (refs[4 * n_tab:5 * n_tab], refs[5 * n_tab:6 * n_tab])
        sems = refs[6 * n_tab:6 * n_tab + 2]

        wid = lax.axis_index("s") * _NC + lax.axis_index("c")
        src0 = src_base + wid * per_w
        dst0 = wid * per_w

        for j in range(n_tab):
            pltpu.sync_copy(idxs[j].at[pl.ds(src0, per_w)], ivs[j])

        def fire(g, s):
            for j in range(n_tab):
                pltpu.async_copy(tabs[j].at[ivs[j].at[pl.ds(g * _C, _C)]],
                                 bufs[s][j], sems[s])

        def drain(s):
            for j in range(n_tab):
                pltpu.make_async_copy(tabs[j].at[pl.ds(0, _C)],
                                      bufs[s][j], sems[s]).wait()

        def scatter(g, s):
            base = dst0 + g * _C
            for j in range(n_tab):
                pltpu.sync_copy(bufs[s][j], outs[j].at[pl.ds(base, _C)])

        fire(0, 0)

        def pair(k, _):
            g = 2 * k
            fire(g + 1, 1)
            drain(0)
            scatter(g, 0)
            fire(g + 2, 0)
            drain(1)
            scatter(g + 1, 1)
            return ()

        lax.fori_loop(0, nchunk // 2 - 1, pair, (), unroll=False)
        g = nchunk - 2
        fire(g + 1, 1)
        drain(0)
        scatter(g, 0)
        drain(1)
        scatter(g + 1, 1)

    return body


def _sc_gather(idxs, tabs, src_base, n_pos):
    n_tab = len(tabs)
    per_w = n_pos // _NW
    mesh = plsc.VectorSubcoreMesh(core_axis_name="c", subcore_axis_name="s")
    row = jax.ShapeDtypeStruct((n_pos, GW), jnp.float32)
    ebuf = pltpu.VMEM((_C, GW), jnp.float32)
    f = pl.kernel(
        _make_sc_body(n_tab, src_base, n_pos),
        mesh=mesh,
        out_type=tuple([row] * n_tab),
        scratch_types=(
            [pltpu.VMEM((per_w,), jnp.int32)] * n_tab
            + [ebuf] * (2 * n_tab)
            + [pltpu.SemaphoreType.DMA, pltpu.SemaphoreType.DMA]
        ),
    )
    out = f(*idxs, *tabs)
    return out if isinstance(out, (tuple, list)) else (out,)


_R = 2048
_NBLK = HALF // _R


def _tc_body(c1_ref, c2_ref, c3_ref, i0_ref, el_ref, du_ref, emb0_ref,
             w0_ref, w_ref, wel_ref, wdu_ref, b_ref, *rest):
    out_ref = rest[-1]
    w = w_ref[...]
    acc = jnp.dot(c1_ref[...], w[0 * GW:1 * GW],
                  preferred_element_type=jnp.float32)
    acc += jnp.dot(c2_ref[...], w[1 * GW:2 * GW],
                   preferred_element_type=jnp.float32)
    acc += jnp.dot(c3_ref[...], w[2 * GW:3 * GW],
                   preferred_element_type=jnp.float32)
    m0 = jnp.dot(emb0_ref[...], w0_ref[...],
                 preferred_element_type=jnp.float32)
    iota8 = lax.broadcasted_iota(jnp.int32, (1, 8), 1)
    oh = jnp.where(i0_ref[...][:, None] == iota8, 1.0, 0.0)
    acc += jnp.dot(oh, m0, preferred_element_type=jnp.float32)
    el = el_ref[...][:, None]
    du = du_ref[...][:, None]
    out_ref[...] = (acc + el * wel_ref[...][None, :] + du * wdu_ref[...][None, :]
                    + b_ref[...][None, :])


def _tc_matmul(half, c1, c2, c3, i0, el, du, emb0, w0, w_mid, w_el, w_du, b,
               xprev=None):
    off = half * _NBLK
    half_spec = pl.BlockSpec((_R, GW), lambda i: (i, 0))
    flat_spec = pl.BlockSpec((_R,), lambda i, off=off: (i + off,))
    in_specs = [
        half_spec, half_spec, half_spec,
        flat_spec, flat_spec, flat_spec,
        pl.BlockSpec((8, INTD), lambda i: (0, 0)),
        pl.BlockSpec((INTD, HD), lambda i: (0, 0)),
        pl.BlockSpec((3 * GW, HD), lambda i: (0, 0)),
        pl.BlockSpec((HD,), lambda i: (0,)),
        pl.BlockSpec((HD,), lambda i: (0,)),
        pl.BlockSpec((HD,), lambda i: (0,)),
    ]
    args = [c1, c2, c3, i0, el, du, emb0, w0, w_mid, w_el, w_du, b]
    aliases = {}
    if xprev is not None:
        in_specs.append(pl.BlockSpec(memory_space=pl.ANY))
        args.append(xprev)
        aliases = {12: 0}
    return pl.pallas_call(
        _tc_body,
        grid=(_NBLK,),
        in_specs=in_specs,
        out_specs=pl.BlockSpec((_R, HD), lambda i, off=off: (i + off, 0)),
        out_shape=jax.ShapeDtypeStruct((BS, HD), jnp.float32),
        input_output_aliases=aliases,
    )(*args)


def kernel(interaction, assessmentItemID, testId, KnowledgeTag, elapsed,
           duration, emb_interaction, emb_assessmentItemID, emb_testId,
           emb_KnowledgeTag, W, b):
    batch_size, seq_len = interaction.shape[0], interaction.shape[1]
    zcol = jnp.zeros((100001, GW - INTD), jnp.float32)
    t1 = jnp.concatenate([emb_assessmentItemID, zcol], axis=1)
    rep = lambda t: jnp.tile(jnp.pad(t, ((0, 1024 - 1001), (0, GW - INTD))),
                             (REP, 1))
    t2 = rep(emb_testId)
    t3 = rep(emb_KnowledgeTag)
    iota = jnp.arange(BS, dtype=jnp.int32)
    spread = (iota & (REP - 1)) << 10
    i1 = assessmentItemID.reshape(-1)
    i2 = testId.reshape(-1) + spread
    i3 = KnowledgeTag.reshape(-1) + spread
    c2a, c3a = _sc_gather((i2, i3), (t2, t3), 0, HALF)
    (c1a,) = _sc_gather((i1,), (t1,), 0, HALF)
    c2b, c3b = _sc_gather((i2, i3), (t2, t3), HALF, HALF)
    (c1b,) = _sc_gather((i1,), (t1,), HALF, HALF)
    emb0 = jnp.pad(emb_interaction, ((0, 8 - 3), (0, 0)))
    w_pad = jnp.concatenate(
        [W[INTD:4 * INTD].reshape(3, INTD, HD),
         jnp.zeros((3, GW - INTD, HD), jnp.float32)], axis=1).reshape(3 * GW, HD)
    i0 = interaction.reshape(-1)
    el = elapsed.reshape(-1)
    du = duration.reshape(-1)
    common = (i0, el, du, emb0, W[:INTD], w_pad,
              W[4 * INTD], W[4 * INTD + 1], b)
    xa = _tc_matmul(0, c1a, c2a, c3a, *common)
    x = _tc_matmul(1, c1b, c2b, c3b, *common, xprev=xa)
    return (x.reshape(batch_size, seq_len, HD), batch_size, seq_len)


# speedup vs baseline: 1.0272x; 1.0007x over previous
import jax
import jax.numpy as jnp
from jax import lax
from jax.experimental import pallas as pl
from jax.experimental.pallas import tpu as pltpu
from jax.experimental.pallas import tpu_sc as plsc

B, S = 1024, 200
BS = B * S
HALF = BS // 2
INTD = 64
GW = 128
HD = 192
REP = 64


_NC, _NS = 2, 16
_NW = _NC * _NS
_C = 64


def _make_sc_body(n_tab, src_base, n_pos):
    per_w = n_pos // _NW
    nchunk = per_w // _C

    def body(*refs):
        idxs = refs[:n_tab]
        tabs = refs[n_tab:2 * n_tab]
        outs = refs[2 * n_tab:3 * n_tab]
        ivs = refs[3 * n_tab:4 * n_tab]
        bufs = (refs[4 * n_tab:5 * n_tab], refs[5 * n_tab:6 * n_tab])
        sems = refs[6 * n_tab:6 * n_tab + 2]

        wid = lax.axis_index("s") * _NC + lax.axis_index("c")
        src0 = src_base + wid * per_w
        dst0 = wid * per_w

        for j in range(n_tab):
            pltpu.sync_copy(idxs[j].at[pl.ds(src0, per_w)], ivs[j])

        t2d = [t.at[0] if len(t.shape) == 3 else t for t in tabs]

        def fire(g, s):
            for j in range(n_tab):
                pltpu.async_copy(t2d[j].at[ivs[j].at[pl.ds(g * _C, _C)]],
                                 bufs[s][j], sems[s])

        def drain(s):
            for j in range(n_tab):
                pltpu.make_async_copy(t2d[j].at[pl.ds(0, _C)],
                                      bufs[s][j], sems[s]).wait()

        def scatter(g, s):
            base = dst0 + g * _C
            for j in range(n_tab):
                pltpu.sync_copy(bufs[s][j], outs[j].at[pl.ds(base, _C)])

        fire(0, 0)

        def pair(k, _):
            g = 2 * k
            fire(g + 1, 1)
            drain(0)
            scatter(g, 0)
            fire(g + 2, 0)
            drain(1)
            scatter(g + 1, 1)
            return ()

        lax.fori_loop(0, nchunk // 2 - 1, pair, (), unroll=False)
        g = nchunk - 2
        fire(g + 1, 1)
        drain(0)
        scatter(g, 0)
        drain(1)
        scatter(g + 1, 1)

    return body


def _sc_gather(idxs, tabs, src_base, n_pos):
    n_tab = len(tabs)
    per_w = n_pos // _NW
    mesh = plsc.VectorSubcoreMesh(core_axis_name="c", subcore_axis_name="s")
    row = jax.ShapeDtypeStruct((n_pos, GW), jnp.float32)
    ebuf = pltpu.VMEM((_C, GW), jnp.float32)
    f = pl.kernel(
        _make_sc_body(n_tab, src_base, n_pos),
        mesh=mesh,
        out_type=tuple([row] * n_tab),
        scratch_types=(
            [pltpu.VMEM((per_w,), jnp.int32)] * n_tab
            + [ebuf] * (2 * n_tab)
            + [pltpu.SemaphoreType.DMA, pltpu.SemaphoreType.DMA]
        ),
    )
    out = f(*idxs, *tabs)
    return out if isinstance(out, (tuple, list)) else (out,)


_R = 2048
_NBLK = HALF // _R


def _tc_body(c1_ref, c2_ref, c3_ref, i0_ref, el_ref, du_ref, emb0_ref,
             w0_ref, w_ref, wel_ref, wdu_ref, b_ref, *rest):
    out_ref = rest[-1]
    w = w_ref[...]
    acc = jnp.dot(c1_ref[...], w[0 * GW:1 * GW],
                  preferred_element_type=jnp.float32)
    acc += jnp.dot(c2_ref[...], w[1 * GW:2 * GW],
                   preferred_element_type=jnp.float32)
    acc += jnp.dot(c3_ref[...], w[2 * GW:3 * GW],
                   preferred_element_type=jnp.float32)
    m0 = jnp.dot(emb0_ref[...], w0_ref[...],
                 preferred_element_type=jnp.float32)
    iota8 = lax.broadcasted_iota(jnp.int32, (1, 8), 1)
    oh = jnp.where(i0_ref[...][:, None] == iota8, 1.0, 0.0)
    acc += jnp.dot(oh, m0, preferred_element_type=jnp.float32)
    el = el_ref[...][:, None]
    du = du_ref[...][:, None]
    out_ref[...] = (acc + el * wel_ref[...][None, :] + du * wdu_ref[...][None, :]
                    + b_ref[...][None, :])


def _tc_matmul(half, c1, c2, c3, i0, el, du, emb0, w0, w_mid, w_el, w_du, b,
               xprev=None):
    off = half * _NBLK
    half_spec = pl.BlockSpec((_R, GW), lambda i: (i, 0))
    flat_spec = pl.BlockSpec((_R,), lambda i, off=off: (i + off,))
    in_specs = [
        half_spec, half_spec, half_spec,
        flat_spec, flat_spec, flat_spec,
        pl.BlockSpec((8, INTD), lambda i: (0, 0)),
        pl.BlockSpec((INTD, HD), lambda i: (0, 0)),
        pl.BlockSpec((3 * GW, HD), lambda i: (0, 0)),
        pl.BlockSpec((HD,), lambda i: (0,)),
        pl.BlockSpec((HD,), lambda i: (0,)),
        pl.BlockSpec((HD,), lambda i: (0,)),
    ]
    args = [c1, c2, c3, i0, el, du, emb0, w0, w_mid, w_el, w_du, b]
    aliases = {}
    if xprev is not None:
        in_specs.append(pl.BlockSpec(memory_space=pl.ANY))
        args.append(xprev)
        aliases = {12: 0}
    return pl.pallas_call(
        _tc_body,
        grid=(_NBLK,),
        in_specs=in_specs,
        out_specs=pl.BlockSpec((_R, HD), lambda i, off=off: (i + off, 0)),
        out_shape=jax.ShapeDtypeStruct((BS, HD), jnp.float32),
        input_output_aliases=aliases,
    )(*args)


def kernel(interaction, assessmentItemID, testId, KnowledgeTag, elapsed,
           duration, emb_interaction, emb_assessmentItemID, emb_testId,
           emb_KnowledgeTag, W, b):
    batch_size, seq_len = interaction.shape[0], interaction.shape[1]
    zcol = jnp.zeros((1, 100001, GW - INTD), jnp.float32)
    t1 = jnp.concatenate([emb_assessmentItemID[None], zcol], axis=2)
    rep = lambda t: jnp.tile(jnp.pad(t, ((0, 1024 - 1001), (0, GW - INTD))),
                             (REP, 1))
    t2 = rep(emb_testId)
    t3 = rep(emb_KnowledgeTag)
    iota = jnp.arange(BS, dtype=jnp.int32)
    spread = (iota & (REP - 1)) << 10
    i1 = assessmentItemID.reshape(-1)
    i2 = testId.reshape(-1) + spread
    i3 = KnowledgeTag.reshape(-1) + spread
    c2a, c3a = _sc_gather((i2, i3), (t2, t3), 0, HALF)
    (c1a,) = _sc_gather((i1,), (t1,), 0, HALF)
    c2b, c3b = _sc_gather((i2, i3), (t2, t3), HALF, HALF)
    (c1b,) = _sc_gather((i1,), (t1,), HALF, HALF)
    emb0 = jnp.pad(emb_interaction, ((0, 8 - 3), (0, 0)))
    w_pad = jnp.concatenate(
        [W[INTD:4 * INTD].reshape(3, INTD, HD),
         jnp.zeros((3, GW - INTD, HD), jnp.float32)], axis=1).reshape(3 * GW, HD)
    i0 = interaction.reshape(-1)
    el = elapsed.reshape(-1)
    du = duration.reshape(-1)
    common = (i0, el, du, emb0, W[:INTD], w_pad,
              W[4 * INTD], W[4 * INTD + 1], b)
    xa = _tc_matmul(0, c1a, c2a, c3a, *common)
    x = _tc_matmul(1, c1b, c2b, c3b, *common, xprev=xa)
    return (x.reshape(batch_size, seq_len, HD), batch_size, seq_len)


# speedup vs baseline: 1.0507x; 1.0229x over previous
import jax
import jax.numpy as jnp
from jax import lax
from jax.experimental import pallas as pl
from jax.experimental.pallas import tpu as pltpu
from jax.experimental.pallas import tpu_sc as plsc

B, S = 1024, 200
BS = B * S
HALF = BS // 2
INTD = 64
GW = 128
HD = 192
REP = 64


_NC, _NS = 2, 16
_NW = _NC * _NS
_C = 64


def _make_sc_body(n_tab, src_base, n_pos):
    per_w = n_pos // _NW
    nchunk = per_w // _C

    def body(*refs):
        idxs = refs[:n_tab]
        tabs = refs[n_tab:2 * n_tab]
        outs = refs[2 * n_tab:3 * n_tab]
        ivs = refs[3 * n_tab:4 * n_tab]
        bufs = (refs[4 * n_tab:5 * n_tab], refs[5 * n_tab:6 * n_tab])
        sems = refs[6 * n_tab:6 * n_tab + 2]

        wid = lax.axis_index("s") * _NC + lax.axis_index("c")
        src0 = src_base + wid * per_w
        dst0 = wid * per_w

        for j in range(n_tab):
            pltpu.sync_copy(idxs[j].at[pl.ds(src0, per_w)], ivs[j])

        t2d = [t.at[0] if len(t.shape) == 3 else t for t in tabs]

        def fire(g, s):
            for j in range(n_tab):
                pltpu.async_copy(t2d[j].at[ivs[j].at[pl.ds(g * _C, _C)]],
                                 bufs[s][j], sems[s])

        def drain(s):
            for j in range(n_tab):
                pltpu.make_async_copy(t2d[j].at[pl.ds(0, _C)],
                                      bufs[s][j], sems[s]).wait()

        def scatter(g, s):
            base = dst0 + g * _C
            for j in range(n_tab):
                pltpu.sync_copy(bufs[s][j], outs[j].at[pl.ds(base, _C)])

        fire(0, 0)

        def pair(k, _):
            g = 2 * k
            fire(g + 1, 1)
            drain(0)
            scatter(g, 0)
            fire(g + 2, 0)
            drain(1)
            scatter(g + 1, 1)
            return ()

        lax.fori_loop(0, nchunk // 2 - 1, pair, (), unroll=False)
        g = nchunk - 2
        fire(g + 1, 1)
        drain(0)
        scatter(g, 0)
        drain(1)
        scatter(g + 1, 1)

    return body


def _sc_gather(idxs, tabs, src_base, n_pos):
    n_tab = len(tabs)
    per_w = n_pos // _NW
    mesh = plsc.VectorSubcoreMesh(core_axis_name="c", subcore_axis_name="s")
    row = jax.ShapeDtypeStruct((n_pos, GW), jnp.float32)
    ebuf = pltpu.VMEM((_C, GW), jnp.float32)
    f = pl.kernel(
        _make_sc_body(n_tab, src_base, n_pos),
        mesh=mesh,
        out_type=tuple([row] * n_tab),
        scratch_types=(
            [pltpu.VMEM((per_w,), jnp.int32)] * n_tab
            + [ebuf] * (2 * n_tab)
            + [pltpu.SemaphoreType.DMA, pltpu.SemaphoreType.DMA]
        ),
    )
    out = f(*idxs, *tabs)
    return out if isinstance(out, (tuple, list)) else (out,)


_R = 2048
_NBLK = HALF // _R


def _tc_body(c1_ref, c2_ref, c3_ref, i0_ref, el_ref, du_ref, emb0_ref,
             w0_ref, w_ref, wel_ref, wdu_ref, b_ref, *rest):
    out_ref = rest[-1]
    w = w_ref[...]
    acc = jnp.dot(c1_ref[...], w[0 * GW:1 * GW],
                  preferred_element_type=jnp.float32)
    acc += jnp.dot(c2_ref[...], w[1 * GW:2 * GW],
                   preferred_element_type=jnp.float32)
    acc += jnp.dot(c3_ref[...], w[2 * GW:3 * GW],
                   preferred_element_type=jnp.float32)
    m0 = jnp.dot(emb0_ref[...], w0_ref[...],
                 preferred_element_type=jnp.float32)
    iota8 = lax.broadcasted_iota(jnp.int32, (1, 8), 1)
    oh = jnp.where(i0_ref[...][:, None] == iota8, 1.0, 0.0)
    acc += jnp.dot(oh, m0, preferred_element_type=jnp.float32)
    el = el_ref[...][:, None]
    du = du_ref[...][:, None]
    out_ref[...] = (acc + el * wel_ref[...][None, :] + du * wdu_ref[...][None, :]
                    + b_ref[...][None, :])


def _tc_matmul(half, c1, c2, c3, i0, el, du, emb0, w0, w_mid, w_el, w_du, b,
               xprev=None):
    off = half * _NBLK
    half_spec = pl.BlockSpec((_R, GW), lambda i: (i, 0))
    flat_spec = pl.BlockSpec((_R,), lambda i, off=off: (i + off,))
    in_specs = [
        half_spec, half_spec, half_spec,
        flat_spec, flat_spec, flat_spec,
        pl.BlockSpec((8, INTD), lambda i: (0, 0)),
        pl.BlockSpec((INTD, HD), lambda i: (0, 0)),
        pl.BlockSpec((3 * GW, HD), lambda i: (0, 0)),
        pl.BlockSpec((HD,), lambda i: (0,)),
        pl.BlockSpec((HD,), lambda i: (0,)),
        pl.BlockSpec((HD,), lambda i: (0,)),
    ]
    args = [c1, c2, c3, i0, el, du, emb0, w0, w_mid, w_el, w_du, b]
    aliases = {}
    if xprev is not None:
        in_specs.append(pl.BlockSpec(memory_space=pl.ANY))
        args.append(xprev)
        aliases = {12: 0}
    return pl.pallas_call(
        _tc_body,
        grid=(_NBLK,),
        in_specs=in_specs,
        out_specs=pl.BlockSpec((_R, HD), lambda i, off=off: (i + off, 0)),
        out_shape=jax.ShapeDtypeStruct((BS, HD), jnp.float32),
        input_output_aliases=aliases,
    )(*args)


def kernel(interaction, assessmentItemID, testId, KnowledgeTag, elapsed,
           duration, emb_interaction, emb_assessmentItemID, emb_testId,
           emb_KnowledgeTag, W, b):
    batch_size, seq_len = interaction.shape[0], interaction.shape[1]
    zcol = jnp.zeros((1, 100001, GW - INTD), jnp.float32)
    t1 = jnp.concatenate([emb_assessmentItemID[None], zcol], axis=2)
    rep = lambda t: jnp.tile(jnp.pad(t, ((0, 1024 - 1001), (0, GW - INTD))),
                             (REP, 1))
    t2 = rep(emb_testId)
    t3 = rep(emb_KnowledgeTag)
    iota = jnp.arange(BS, dtype=jnp.int32)
    spread = (iota & (REP - 1)) << 10
    i1 = assessmentItemID.reshape(-1)
    i2 = testId.reshape(-1) + spread
    i3 = KnowledgeTag.reshape(-1) + spread
    c1a, c2a, c3a = _sc_gather((i1, i2, i3), (t1, t2, t3), 0, HALF)
    c1b, c2b, c3b = _sc_gather((i1, i2, i3), (t1, t2, t3), HALF, HALF)
    emb0 = jnp.pad(emb_interaction, ((0, 8 - 3), (0, 0)))
    w_pad = jnp.concatenate(
        [W[INTD:4 * INTD].reshape(3, INTD, HD),
         jnp.zeros((3, GW - INTD, HD), jnp.float32)], axis=1).reshape(3 * GW, HD)
    i0 = interaction.reshape(-1)
    el = elapsed.reshape(-1)
    du = duration.reshape(-1)
    common = (i0, el, du, emb0, W[:INTD], w_pad,
              W[4 * INTD], W[4 * INTD + 1], b)
    xa = _tc_matmul(0, c1a, c2a, c3a, *common)
    x = _tc_matmul(1, c1b, c2b, c3b, *common, xprev=xa)
    return (x.reshape(batch_size, seq_len, HD), batch_size, seq_len)
